# baseline (speedup 1.0000x reference)
import functools

import jax
import jax.numpy as jnp
from jax import lax
from jax.experimental import pallas as pl
from jax.experimental.pallas import tpu as pltpu
from jax.experimental.pallas import tpu_sc as plsc

RADIUS2 = 0.25 * 0.25
KNN = 10
NCAND = 24
NPTS = 100000
PBLK = 2048
NBLK = 49
SUB = 7
W = 256
NPAD = NBLK * PBLK
QTOT = 4096
QBLK = 256
BIGI = 2**30
INF = float("inf")
ZERO_ROW = NPTS
IDX_MASK = 0x1FFFF
KEY_MAX = 0x7FFFFFFF


def _sweep_body(qx_ref, qy_ref, qz_ref, p_ref, idx_ref, gidx_ref):
    qx = qx_ref[...]
    qy = qy_ref[...]
    qz = qz_ref[...]
    lane_w = lax.broadcasted_iota(jnp.int32, (QBLK, W), 1)

    def block(b, carry):
        k1, k2, k3, k4 = carry
        for f in range(PBLK // W):
            px = p_ref[b, 0:1, pl.ds(f * W, W)]
            py = p_ref[b, 1:2, pl.ds(f * W, W)]
            pz = p_ref[b, 2:3, pl.ds(f * W, W)]
            dx = qx - px
            dy = qy - py
            dz = qz - pz
            c = (dx * dx + dy * dy) + dz * dz
            kb = lax.bitcast_convert_type(c, jnp.int32) & ~IDX_MASK
            k = kb | (lane_w + (b * PBLK + f * W))
            l1 = k < k1
            l2 = k < k2
            l3 = k < k3
            l4 = k < k4
            k4 = jnp.where(l3, k3, jnp.where(l4, k, k4))
            k3 = jnp.where(l2, k2, jnp.where(l3, k, k3))
            k2 = jnp.where(l1, k1, jnp.where(l2, k, k2))
            k1 = jnp.where(l1, k, k1)
        return k1, k2, k3, k4

    kI = jnp.full((QBLK, W), KEY_MAX, jnp.int32)
    carry = (kI, kI, kI, kI)
    for b in range(NBLK):
        carry = block(b, carry)
    k1, k2, k3, k4 = carry

    keys_all = jnp.concatenate([k1, k2, k3, k4], axis=1)
    picks = []
    for _ in range(NCAND):
        m = jnp.min(keys_all, axis=1, keepdims=True)
        picks.append(m)
        keys_all = jnp.where(keys_all == m, KEY_MAX, keys_all)

    cand = jnp.concatenate(picks, axis=1)
    raw = cand & IDX_MASK
    idx_ref[...] = raw
    gidx_ref[...] = jnp.where(raw < NPTS, raw, ZERO_ROW)


def _sweep(qx, qy, qz, pblocks):
    grid = QTOT // QBLK
    return pl.pallas_call(
        _sweep_body,
        grid=(grid,),
        in_specs=[
            pl.BlockSpec((QBLK, 1), lambda i: (i, 0)),
            pl.BlockSpec((QBLK, 1), lambda i: (i, 0)),
            pl.BlockSpec((QBLK, 1), lambda i: (i, 0)),
            pl.BlockSpec((NBLK, 3, PBLK), lambda i: (0, 0, 0)),
        ],
        out_specs=[
            pl.BlockSpec((QBLK, NCAND), lambda i: (i, 0)),
            pl.BlockSpec((QBLK, NCAND), lambda i: (i, 0)),
        ],
        out_shape=[
            jax.ShapeDtypeStruct((QTOT, NCAND), jnp.int32),
            jax.ShapeDtypeStruct((QTOT, NCAND), jnp.int32),
        ],
        compiler_params=pltpu.CompilerParams(
            dimension_semantics=("arbitrary",),
        ),
    )(qx, qy, qz, pblocks)


def _make_sc_gather(row_w, batch):
    info = plsc.get_sparse_core_info()
    nw = info.num_cores * info.num_subcores
    b_per_w = batch // nw
    chunk = 128
    n_chunks = b_per_w // chunk
    mesh = plsc.VectorSubcoreMesh(core_axis_name="c", subcore_axis_name="s")

    @functools.partial(
        pl.kernel,
        mesh=mesh,
        out_type=jax.ShapeDtypeStruct((batch, row_w), jnp.float32),
        scratch_types=[
            pltpu.VMEM((chunk,), jnp.int32),
            pltpu.VMEM((chunk, row_w), jnp.float32),
            pltpu.SemaphoreType.DMA,
        ],
    )
    def gather_k(table_hbm, idx_hbm, out_hbm, idx_v, rows_v, sem):
        wid = lax.axis_index("s") * info.num_cores + lax.axis_index("c")
        for c in range(n_chunks):
            base = wid * b_per_w + c * chunk
            pltpu.sync_copy(idx_hbm.at[pl.ds(base, chunk)], idx_v)
            pltpu.async_copy(table_hbm.at[idx_v], rows_v, sem).wait()
            pltpu.sync_copy(rows_v, out_hbm.at[pl.ds(base, chunk)])

    return gather_k


def _rerank_body(
    qx_ref, qy_ref, qz_ref, cx_ref, cy_ref, cz_ref, ci_ref,
    map_ref, ox_ref, oy_ref, oz_ref
):
    qx = qx_ref[...]
    qy = qy_ref[...]
    qz = qz_ref[...]
    cx = cx_ref[...]
    cy = cy_ref[...]
    cz = cz_ref[...]
    ci = ci_ref[...]
    dx = qx - cx
    dy = qy - cy
    dz = qz - cz
    d2 = (dx * dx + dy * dy) + dz * dz
    d2 = jnp.where(ci < NPTS, d2, INF)

    vals, idxs, oxs, oys, ozs = [], [], [], [], []
    for _ in range(KNN):
        m = jnp.min(d2, axis=1, keepdims=True)
        sel = d2 == m
        pick = jnp.min(jnp.where(sel, ci, BIGI), axis=1, keepdims=True)
        hit = sel & (ci == pick)
        vals.append(m)
        idxs.append(pick)
        oxs.append(jnp.sum(jnp.where(hit, cx, 0.0), axis=1, keepdims=True))
        oys.append(jnp.sum(jnp.where(hit, cy, 0.0), axis=1, keepdims=True))
        ozs.append(jnp.sum(jnp.where(hit, cz, 0.0), axis=1, keepdims=True))
        d2 = jnp.where(hit, INF, d2)

    v = jnp.concatenate(vals, axis=1)
    valid = v <= RADIUS2
    map_ref[...] = jnp.where(valid, jnp.concatenate(idxs, axis=1), 0)
    ox_ref[...] = jnp.where(valid, jnp.concatenate(oxs, axis=1), 0.0)
    oy_ref[...] = jnp.where(valid, jnp.concatenate(oys, axis=1), 0.0)
    oz_ref[...] = jnp.where(valid, jnp.concatenate(ozs, axis=1), 0.0)


def _rerank(qx, qy, qz, cx, cy, cz, ci):
    grid = QTOT // QBLK
    qspec = pl.BlockSpec((QBLK, 1), lambda i: (i, 0))
    cspec = pl.BlockSpec((QBLK, NCAND), lambda i: (i, 0))
    ospec = pl.BlockSpec((QBLK, KNN), lambda i: (i, 0))
    return pl.pallas_call(
        _rerank_body,
        grid=(grid,),
        in_specs=[qspec, qspec, qspec, cspec, cspec, cspec, cspec],
        out_specs=[ospec, ospec, ospec, ospec],
        out_shape=[
            jax.ShapeDtypeStruct((QTOT, KNN), jnp.int32),
            jax.ShapeDtypeStruct((QTOT, KNN), jnp.float32),
            jax.ShapeDtypeStruct((QTOT, KNN), jnp.float32),
            jax.ShapeDtypeStruct((QTOT, KNN), jnp.float32),
        ],
        compiler_params=pltpu.CompilerParams(
            dimension_semantics=("arbitrary",),
        ),
    )(qx, qy, qz, cx, cy, cz, ci)


ROW_W = 128
TAB_ROWS = NPTS + 8


def kernel(x, p_grid):
    pts = x[0]
    pg = p_grid.reshape(1, -1, 3)[0]

    ppad = jnp.pad(pts, ((0, NPAD - NPTS), (0, 0)), constant_values=2.0)
    pblocks = ppad.T.reshape(3, NBLK, PBLK).transpose(1, 0, 2)

    qx = pg[:, 0:1]
    qy = pg[:, 1:2]
    qz = pg[:, 2:3]

    raw_idx, gidx = _sweep(qx, qy, qz, pblocks)

    table = jnp.pad(pts, ((0, TAB_ROWS - NPTS), (0, ROW_W - 3)))
    gathered = _make_sc_gather(ROW_W, QTOT * NCAND)(table, gidx.reshape(-1))
    g = gathered.reshape(QTOT, NCAND, ROW_W)
    cx = g[:, :, 0]
    cy = g[:, :, 1]
    cz = g[:, :, 2]

    mapping, ox, oy, oz = _rerank(qx, qy, qz, cx, cy, cz, raw_idx)
    outputs = jnp.stack([ox, oy, oz], axis=-1)

    return mapping[None], outputs[None]
